# Initial kernel scaffold; baseline (speedup 1.0000x reference)
import functools

import jax
import jax.numpy as jnp
from jax import lax
from jax.experimental import pallas as pl
from jax.experimental.pallas import tpu as pltpu
from jax.experimental.pallas import tpu_sc as plsc

_NC = 2
_NS = 16
_NW = _NC * _NS
_LANES = 16
_CHUNK = 128
_ZROWS = 64


def _sc_degree(col3, n_pad, cpt):
    rpt = n_pad // _NS
    mesh = plsc.VectorSubcoreMesh(core_axis_name="c", subcore_axis_name="s")

    @functools.partial(
        pl.kernel,
        out_type=jax.ShapeDtypeStruct((_NC, n_pad, _LANES), jnp.float32),
        mesh=mesh,
        scratch_types=[
            pltpu.VMEM((cpt, _CHUNK), jnp.int32),
            pltpu.VMEM((_CHUNK, _LANES), jnp.float32),
            pltpu.VMEM((rpt, _LANES), jnp.float32),
            pltpu.VMEM_SHARED((n_pad, _LANES), jnp.float32),
        ],
    )
    def deg_kernel(col_hbm, out_hbm, idx_v, ones_v, zero_v, acc_sh):
        c = lax.axis_index("c")
        s = lax.axis_index("s")
        wid = c * _NS + s

        @pl.loop(0, _CHUNK)
        def _(i):
            ones_v[i, :] = jnp.ones((_LANES,), jnp.float32)

        @pl.loop(0, rpt)
        def _(i):
            zero_v[i, :] = jnp.zeros((_LANES,), jnp.float32)

        pltpu.sync_copy(col_hbm.at[wid], idx_v)
        pltpu.sync_copy(zero_v, acc_sh.at[pl.ds(s * rpt, rpt)])
        plsc.subcore_barrier()

        @pl.loop(0, cpt)
        def _(j):
            pltpu.sync_copy(ones_v, acc_sh.at[idx_v.at[j]], add=True)

        plsc.subcore_barrier()
        pltpu.sync_copy(acc_sh.at[pl.ds(s * rpt, rpt)],
                        out_hbm.at[c, pl.ds(s * rpt, rpt)])

    return deg_kernel(col3)


def _sc_scatter(y, row3, col3, n_pad, cpt, d):
    rpt = n_pad // _NS
    mesh = plsc.VectorSubcoreMesh(core_axis_name="c", subcore_axis_name="s")

    @functools.partial(
        pl.kernel,
        out_type=jax.ShapeDtypeStruct((_NC, n_pad, d), jnp.float32),
        mesh=mesh,
        scratch_types=[
            pltpu.VMEM((cpt, _CHUNK), jnp.int32),
            pltpu.VMEM((cpt, _CHUNK), jnp.int32),
            pltpu.VMEM((_CHUNK, d), jnp.float32),
            pltpu.VMEM((_ZROWS, d), jnp.float32),
            pltpu.VMEM_SHARED((n_pad, d), jnp.float32),
        ],
    )
    def scat_kernel(y_hbm, row_hbm, col_hbm, out_hbm,
                    ridx, cidx, msg, zbuf, acc_sh):
        c = lax.axis_index("c")
        s = lax.axis_index("s")
        wid = c * _NS + s

        pltpu.sync_copy(row_hbm.at[wid], ridx)
        pltpu.sync_copy(col_hbm.at[wid], cidx)

        @pl.loop(0, _ZROWS)
        def _(i):
            @pl.loop(0, d, step=_LANES)
            def _(jj):
                zbuf[i, pl.ds(jj, _LANES)] = jnp.zeros((_LANES,), jnp.float32)

        @pl.loop(0, rpt, step=_ZROWS)
        def _(r):
            pltpu.sync_copy(zbuf, acc_sh.at[pl.ds(s * rpt + r, _ZROWS)])

        plsc.subcore_barrier()

        @pl.loop(0, cpt)
        def _(j):
            pltpu.sync_copy(y_hbm.at[ridx.at[j]], msg)
            pltpu.sync_copy(msg, acc_sh.at[cidx.at[j]], add=True)

        plsc.subcore_barrier()
        pltpu.sync_copy(acc_sh.at[pl.ds(s * rpt, rpt)],
                        out_hbm.at[c, pl.ds(s * rpt, rpt)])

    return scat_kernel(y, row3, col3)


def _tc_matmul(x, w):
    n, d = x.shape
    blk = 1000

    def body(x_ref, w_ref, o_ref):
        o_ref[...] = jnp.dot(x_ref[...], w_ref[...],
                             preferred_element_type=jnp.float32)

    return pl.pallas_call(
        body,
        grid=(n // blk,),
        in_specs=[pl.BlockSpec((blk, d), lambda i: (i, 0)),
                  pl.BlockSpec((d, d), lambda i: (0, 0))],
        out_specs=pl.BlockSpec((blk, d), lambda i: (i, 0)),
        out_shape=jax.ShapeDtypeStruct((n, d), jnp.float32),
    )(x, w)


def _tc_scale(xw, degp):
    n, d = xw.shape
    blk = 1000

    def body(x_ref, dp_ref, o_ref):
        deg = dp_ref[0] + dp_ref[1] + 1.0
        dis = lax.rsqrt(deg)[:, 0:1]
        o_ref[...] = x_ref[...] * dis

    return pl.pallas_call(
        body,
        grid=(n // blk,),
        in_specs=[pl.BlockSpec((blk, d), lambda i: (i, 0)),
                  pl.BlockSpec((_NC, blk, _LANES), lambda i: (0, i, 0))],
        out_specs=pl.BlockSpec((blk, d), lambda i: (i, 0)),
        out_shape=jax.ShapeDtypeStruct((n, d), jnp.float32),
    )(xw, degp)


def _tc_finalize(accp, y, degp, b, bn_weight, bn_bias, bn_mean, bn_var):
    n, d = y.shape
    blk = 1000
    p2 = lambda a: a.reshape(1, d)

    def body(a_ref, y_ref, dp_ref, b_ref, w_ref, bb_ref, m_ref, v_ref, o_ref):
        deg = dp_ref[0] + dp_ref[1] + 1.0
        dis = lax.rsqrt(deg)[:, 0:1]
        t = (a_ref[0] + a_ref[1] + y_ref[...]) * dis + b_ref[...]
        t = jnp.maximum(t, 0.0)
        scale = w_ref[...] * lax.rsqrt(v_ref[...] + 1e-5)
        o_ref[...] = (t - m_ref[...]) * scale + bb_ref[...]

    vspec = pl.BlockSpec((1, d), lambda i: (0, 0))
    return pl.pallas_call(
        body,
        grid=(n // blk,),
        in_specs=[pl.BlockSpec((_NC, blk, d), lambda i: (0, i, 0)),
                  pl.BlockSpec((blk, d), lambda i: (i, 0)),
                  pl.BlockSpec((_NC, blk, _LANES), lambda i: (0, i, 0)),
                  vspec, vspec, vspec, vspec, vspec],
        out_specs=pl.BlockSpec((blk, d), lambda i: (i, 0)),
        out_shape=jax.ShapeDtypeStruct((n, d), jnp.float32),
    )(accp, y, degp, p2(b), p2(bn_weight), p2(bn_bias), p2(bn_mean),
      p2(bn_var))


def kernel(node_features, edge_indices, W, b, bn_weight, bn_bias, bn_mean,
           bn_var):
    n, d = node_features.shape
    e = edge_indices.shape[1]

    rpt = -(-n // _NS)
    rpt += (-rpt) % _ZROWS
    n_pad = _NS * rpt

    cpt = -(-e // (_NW * _CHUNK))
    e_pad = _NW * _CHUNK * cpt
    pad = e_pad - e

    row = edge_indices[0].astype(jnp.int32)
    col = edge_indices[1].astype(jnp.int32)
    row_p = jnp.concatenate([row, jnp.zeros((pad,), jnp.int32)])
    pad_cols = n + (jnp.arange(pad, dtype=jnp.int32) % (n_pad - n))
    col_p = jnp.concatenate([col, pad_cols])
    row3 = row_p.reshape(_NW, cpt, _CHUNK)
    col3 = col_p.reshape(_NW, cpt, _CHUNK)

    degp = _sc_degree(col3, n_pad, cpt)
    xw = _tc_matmul(node_features, W)
    y = _tc_scale(xw, degp)
    accp = _sc_scatter(y, row3, col3, n_pad, cpt, d)
    return _tc_finalize(accp, y, degp, b, bn_weight, bn_bias, bn_mean,
                        bn_var)


# speedup vs baseline: 14.1120x; 14.1120x over previous
import dataclasses
import functools

import jax
import jax.numpy as jnp
from jax import lax
from jax.experimental import pallas as pl
from jax.experimental.pallas import tpu as pltpu
from jax.experimental.pallas import tpu_sc as plsc

_NC = 2
_NS = 16
_NW = _NC * _NS
_LANES = 16
_CHUNK = 128
_ZROWS = 64

_MESH = plsc.VectorSubcoreMesh(core_axis_name="c", subcore_axis_name="s")
_CP_NO_LAYOUT = dataclasses.replace(
    pltpu.CompilerParams(), needs_layout_passes=False)


def _sc_degree(col3, n_pad, cpt):
    @functools.partial(
        pl.kernel,
        out_type=jax.ShapeDtypeStruct((_NW, n_pad), jnp.float32),
        mesh=_MESH,
        compiler_params=_CP_NO_LAYOUT,
        scratch_types=[
            pltpu.VMEM((cpt, _CHUNK), jnp.int32),
            pltpu.VMEM((n_pad,), jnp.float32),
        ],
    )
    def deg_kernel(col_hbm, out_hbm, idx_v, hist_v):
        c = lax.axis_index("c")
        s = lax.axis_index("s")
        wid = c * _NS + s
        pltpu.sync_copy(col_hbm.at[wid], idx_v)

        @pl.loop(0, n_pad, step=_LANES)
        def _(i):
            hist_v[pl.ds(i, _LANES)] = jnp.zeros((_LANES,), jnp.float32)

        ones = jnp.ones((_LANES,), jnp.float32)

        @pl.loop(0, cpt)
        def _(j):
            @pl.loop(0, _CHUNK, step=_LANES)
            def _(q):
                iv = idx_v[j, pl.ds(q, _LANES)]
                plsc.addupdate_scatter(hist_v, [iv], ones)

        pltpu.sync_copy(hist_v, out_hbm.at[wid])

    return deg_kernel(col3)


def _sc_scatter(y, row3, col3, n_pad, cpt, d):
    rpt = n_pad // _NS

    @functools.partial(
        pl.kernel,
        out_type=jax.ShapeDtypeStruct((_NC, n_pad, d), jnp.float32),
        mesh=_MESH,
        scratch_types=[
            pltpu.VMEM((cpt, _CHUNK), jnp.int32),
            pltpu.VMEM((cpt, _CHUNK), jnp.int32),
            pltpu.VMEM((_CHUNK, d), jnp.float32),
            pltpu.VMEM((_ZROWS, d), jnp.float32),
            pltpu.VMEM_SHARED((n_pad, d), jnp.float32),
        ],
    )
    def scat_kernel(y_hbm, row_hbm, col_hbm, out_hbm,
                    ridx, cidx, msg, zbuf, acc_sh):
        c = lax.axis_index("c")
        s = lax.axis_index("s")
        wid = c * _NS + s

        pltpu.sync_copy(row_hbm.at[wid], ridx)
        pltpu.sync_copy(col_hbm.at[wid], cidx)

        @pl.loop(0, _ZROWS)
        def _(i):
            @pl.loop(0, d, step=_LANES)
            def _(q):
                zbuf[i, pl.ds(q, _LANES)] = jnp.zeros((_LANES,), jnp.float32)

        @pl.loop(0, rpt, step=_ZROWS)
        def _(r):
            pltpu.sync_copy(zbuf, acc_sh.at[pl.ds(s * rpt + r, _ZROWS)])

        plsc.subcore_barrier()

        @pl.loop(0, cpt)
        def _(j):
            pltpu.sync_copy(y_hbm.at[ridx.at[j]], msg)
            pltpu.sync_copy(msg, acc_sh.at[cidx.at[j]], add=True)

        plsc.subcore_barrier()
        pltpu.sync_copy(acc_sh.at[pl.ds(s * rpt, rpt)],
                        out_hbm.at[c, pl.ds(s * rpt, rpt)])

    return scat_kernel(y, row3, col3)


_BLK = 1024


def _tc_matmul(x, w):
    n, d = x.shape
    blk = _BLK

    def body(x_ref, w_ref, o_ref):
        o_ref[...] = jnp.dot(x_ref[...], w_ref[...],
                             preferred_element_type=jnp.float32)

    return pl.pallas_call(
        body,
        grid=(n // blk,),
        in_specs=[pl.BlockSpec((blk, d), lambda i: (i, 0)),
                  pl.BlockSpec((d, d), lambda i: (0, 0))],
        out_specs=pl.BlockSpec((blk, d), lambda i: (i, 0)),
        out_shape=jax.ShapeDtypeStruct((n, d), jnp.float32),
    )(x, w)


def _tc_scale(xw, degp):
    n, d = xw.shape
    blk = _BLK

    def body(x_ref, dp_ref, o_ref):
        deg = jnp.sum(dp_ref[...], axis=0) + 1.0
        dis = lax.rsqrt(deg)[:, None]
        o_ref[...] = x_ref[...] * dis

    return pl.pallas_call(
        body,
        grid=(n // blk,),
        in_specs=[pl.BlockSpec((blk, d), lambda i: (i, 0)),
                  pl.BlockSpec((_NW, blk), lambda i: (0, i))],
        out_specs=pl.BlockSpec((blk, d), lambda i: (i, 0)),
        out_shape=jax.ShapeDtypeStruct((n, d), jnp.float32),
    )(xw, degp)


def _tc_finalize(accp, y, degp, b, bn_weight, bn_bias, bn_mean, bn_var):
    n, d = y.shape
    blk = _BLK
    p2 = lambda a: a.reshape(1, d)

    def body(a_ref, y_ref, dp_ref, b_ref, w_ref, bb_ref, m_ref, v_ref, o_ref):
        deg = jnp.sum(dp_ref[...], axis=0) + 1.0
        dis = lax.rsqrt(deg)[:, None]
        t = (a_ref[0] + a_ref[1] + y_ref[...]) * dis + b_ref[...]
        t = jnp.maximum(t, 0.0)
        scale = w_ref[...] * lax.rsqrt(v_ref[...] + 1e-5)
        o_ref[...] = (t - m_ref[...]) * scale + bb_ref[...]

    vspec = pl.BlockSpec((1, d), lambda i: (0, 0))
    return pl.pallas_call(
        body,
        grid=(n // blk,),
        in_specs=[pl.BlockSpec((_NC, blk, d), lambda i: (0, i, 0)),
                  pl.BlockSpec((blk, d), lambda i: (i, 0)),
                  pl.BlockSpec((_NW, blk), lambda i: (0, i)),
                  vspec, vspec, vspec, vspec, vspec],
        out_specs=pl.BlockSpec((blk, d), lambda i: (i, 0)),
        out_shape=jax.ShapeDtypeStruct((n, d), jnp.float32),
    )(accp, y, degp, p2(b), p2(bn_weight), p2(bn_bias), p2(bn_mean),
      p2(bn_var))


def kernel(node_features, edge_indices, W, b, bn_weight, bn_bias, bn_mean,
           bn_var):
    n, d = node_features.shape
    e = edge_indices.shape[1]

    rpt = -(-n // _NS)
    rpt += (-rpt) % _ZROWS
    n_pad = _NS * rpt

    cpt = -(-e // (_NW * _CHUNK))
    cpt += cpt % 2
    e_pad = _NW * _CHUNK * cpt
    pad = e_pad - e

    row = edge_indices[0].astype(jnp.int32)
    col = edge_indices[1].astype(jnp.int32)
    row_p = jnp.concatenate([row, jnp.zeros((pad,), jnp.int32)])
    pad_cols = n + (jnp.arange(pad, dtype=jnp.int32) % (n_pad - n))
    col_p = jnp.concatenate([col, pad_cols])
    row3 = row_p.reshape(_NW, cpt, _CHUNK)
    col3 = col_p.reshape(_NW, cpt, _CHUNK)

    x_pad = jnp.pad(node_features, ((0, n_pad - n), (0, 0)))

    degp = _sc_degree(col3, n_pad, cpt)
    xw = _tc_matmul(x_pad, W)
    y = _tc_scale(xw, degp)
    accp = _sc_scatter(y, row3, col3, n_pad, cpt, d)
    h = _tc_finalize(accp, y, degp, b, bn_weight, bn_bias, bn_mean, bn_var)
    return h[:n]


# speedup vs baseline: 15.6171x; 1.1067x over previous
import dataclasses
import functools

import jax
import jax.numpy as jnp
from jax import lax
from jax.experimental import pallas as pl
from jax.experimental.pallas import tpu as pltpu
from jax.experimental.pallas import tpu_sc as plsc

_NC = 2
_NS = 16
_NW = _NC * _NS
_LANES = 16
_CHUNK = 128
_ZROWS = 64

_MESH = plsc.VectorSubcoreMesh(core_axis_name="c", subcore_axis_name="s")
_CP_NO_LAYOUT = dataclasses.replace(
    pltpu.CompilerParams(), needs_layout_passes=False)


def _sc_degree(col3, n_pad, cpt):
    @functools.partial(
        pl.kernel,
        out_type=jax.ShapeDtypeStruct((_NW, n_pad), jnp.float32),
        mesh=_MESH,
        compiler_params=_CP_NO_LAYOUT,
        scratch_types=[
            pltpu.VMEM((cpt, _CHUNK), jnp.int32),
            pltpu.VMEM((n_pad,), jnp.float32),
        ],
    )
    def deg_kernel(col_hbm, out_hbm, idx_v, hist_v):
        c = lax.axis_index("c")
        s = lax.axis_index("s")
        wid = c * _NS + s
        pltpu.sync_copy(col_hbm.at[wid], idx_v)

        @pl.loop(0, n_pad, step=_LANES)
        def _(i):
            hist_v[pl.ds(i, _LANES)] = jnp.zeros((_LANES,), jnp.float32)

        ones = jnp.ones((_LANES,), jnp.float32)

        @pl.loop(0, cpt)
        def _(j):
            @pl.loop(0, _CHUNK, step=_LANES)
            def _(q):
                iv = idx_v[j, pl.ds(q, _LANES)]
                plsc.addupdate_scatter(hist_v, [iv], ones)

        pltpu.sync_copy(hist_v, out_hbm.at[wid])

    return deg_kernel(col3)


_NBUF = 2
_SB = 16


def _sc_scatter(y, row3, col3, n_pad, cpt, d):
    rpt = n_pad // _NS
    assert cpt % _SB == 0 and _SB % _NBUF == 0

    @functools.partial(
        pl.kernel,
        out_type=jax.ShapeDtypeStruct((_NC, n_pad, d), jnp.float32),
        mesh=_MESH,
        scratch_types=[
            pltpu.VMEM((_SB, _CHUNK), jnp.int32),
            pltpu.VMEM((_SB, _CHUNK), jnp.int32),
            [pltpu.VMEM((_CHUNK, d), jnp.float32) for _ in range(_NBUF)],
            [pltpu.SemaphoreType.DMA for _ in range(_NBUF)],
            pltpu.VMEM((_LANES, d), jnp.float32),
            pltpu.VMEM_SHARED((n_pad, d), jnp.float32),
        ],
    )
    def scat_kernel(y_hbm, row_hbm, col_hbm, out_hbm,
                    ridx, cidx, msgs, sems, zbuf, acc_sh):
        c = lax.axis_index("c")
        s = lax.axis_index("s")
        wid = c * _NS + s

        @pl.loop(0, _LANES)
        def _(i):
            @pl.loop(0, d, step=_LANES)
            def _(q):
                zbuf[i, pl.ds(q, _LANES)] = jnp.zeros((_LANES,), jnp.float32)

        @pl.loop(0, rpt, step=_LANES)
        def _(r):
            pltpu.sync_copy(zbuf, acc_sh.at[pl.ds(s * rpt + r, _LANES)])

        plsc.subcore_barrier()

        @pl.loop(0, cpt, step=_SB)
        def _(jb):
            pltpu.sync_copy(row_hbm.at[wid, pl.ds(jb, _SB)], ridx)
            pltpu.sync_copy(col_hbm.at[wid, pl.ds(jb, _SB)], cidx)
            for b in range(_NBUF):
                pltpu.async_copy(y_hbm.at[ridx.at[b]], msgs[b], sems[b])

            @pl.loop(0, _SB, step=_NBUF)
            def _(q):
                for b in range(_NBUF):
                    pltpu.make_async_copy(
                        y_hbm.at[ridx.at[q + b]], msgs[b], sems[b]).wait()
                    pltpu.sync_copy(msgs[b], acc_sh.at[cidx.at[q + b]],
                                    add=True)

                    @pl.when(q + b + _NBUF < _SB)
                    def _():
                        pltpu.async_copy(
                            y_hbm.at[ridx.at[q + b + _NBUF]], msgs[b], sems[b])

        plsc.subcore_barrier()
        pltpu.sync_copy(acc_sh.at[pl.ds(s * rpt, rpt)],
                        out_hbm.at[c, pl.ds(s * rpt, rpt)])

    return scat_kernel(y, row3, col3)


_BLK = 1024


def _tc_matmul(x, w):
    n, d = x.shape
    blk = _BLK

    def body(x_ref, w_ref, o_ref):
        o_ref[...] = jnp.dot(x_ref[...], w_ref[...],
                             preferred_element_type=jnp.float32)

    return pl.pallas_call(
        body,
        grid=(n // blk,),
        in_specs=[pl.BlockSpec((blk, d), lambda i: (i, 0)),
                  pl.BlockSpec((d, d), lambda i: (0, 0))],
        out_specs=pl.BlockSpec((blk, d), lambda i: (i, 0)),
        out_shape=jax.ShapeDtypeStruct((n, d), jnp.float32),
    )(x, w)


def _tc_scale(xw, degp):
    n, d = xw.shape
    blk = _BLK

    def body(x_ref, dp_ref, o_ref):
        deg = jnp.sum(dp_ref[...], axis=0) + 1.0
        dis = lax.rsqrt(deg)[:, None]
        o_ref[...] = x_ref[...] * dis

    return pl.pallas_call(
        body,
        grid=(n // blk,),
        in_specs=[pl.BlockSpec((blk, d), lambda i: (i, 0)),
                  pl.BlockSpec((_NW, blk), lambda i: (0, i))],
        out_specs=pl.BlockSpec((blk, d), lambda i: (i, 0)),
        out_shape=jax.ShapeDtypeStruct((n, d), jnp.float32),
    )(xw, degp)


def _tc_finalize(accp, y, degp, b, bn_weight, bn_bias, bn_mean, bn_var):
    n, d = y.shape
    blk = _BLK
    p2 = lambda a: a.reshape(1, d)

    def body(a_ref, y_ref, dp_ref, b_ref, w_ref, bb_ref, m_ref, v_ref, o_ref):
        deg = jnp.sum(dp_ref[...], axis=0) + 1.0
        dis = lax.rsqrt(deg)[:, None]
        t = (a_ref[0] + a_ref[1] + y_ref[...]) * dis + b_ref[...]
        t = jnp.maximum(t, 0.0)
        scale = w_ref[...] * lax.rsqrt(v_ref[...] + 1e-5)
        o_ref[...] = (t - m_ref[...]) * scale + bb_ref[...]

    vspec = pl.BlockSpec((1, d), lambda i: (0, 0))
    return pl.pallas_call(
        body,
        grid=(n // blk,),
        in_specs=[pl.BlockSpec((_NC, blk, d), lambda i: (0, i, 0)),
                  pl.BlockSpec((blk, d), lambda i: (i, 0)),
                  pl.BlockSpec((_NW, blk), lambda i: (0, i)),
                  vspec, vspec, vspec, vspec, vspec],
        out_specs=pl.BlockSpec((blk, d), lambda i: (i, 0)),
        out_shape=jax.ShapeDtypeStruct((n, d), jnp.float32),
    )(accp, y, degp, p2(b), p2(bn_weight), p2(bn_bias), p2(bn_mean),
      p2(bn_var))


def kernel(node_features, edge_indices, W, b, bn_weight, bn_bias, bn_mean,
           bn_var):
    n, d = node_features.shape
    e = edge_indices.shape[1]

    rpt = -(-n // _NS)
    rpt += (-rpt) % _ZROWS
    n_pad = _NS * rpt

    cpt = -(-e // (_NW * _CHUNK))
    cpt += (-cpt) % _SB
    e_pad = _NW * _CHUNK * cpt
    pad = e_pad - e

    row = edge_indices[0].astype(jnp.int32)
    col = edge_indices[1].astype(jnp.int32)
    row_p = jnp.concatenate([row, jnp.zeros((pad,), jnp.int32)])
    pad_cols = n + (jnp.arange(pad, dtype=jnp.int32) % (n_pad - n))
    col_p = jnp.concatenate([col, pad_cols])
    row3 = row_p.reshape(_NW, cpt, _CHUNK)
    col3 = col_p.reshape(_NW, cpt, _CHUNK)

    x_pad = jnp.pad(node_features, ((0, n_pad - n), (0, 0)))

    degp = _sc_degree(col3, n_pad, cpt)
    xw = _tc_matmul(x_pad, W)
    y = _tc_scale(xw, degp)
    accp = _sc_scatter(y, row3, col3, n_pad, cpt, d)
    h = _tc_finalize(accp, y, degp, b, bn_weight, bn_bias, bn_mean, bn_var)
    return h[:n]


# speedup vs baseline: 15.6205x; 1.0002x over previous
import dataclasses
import functools

import jax
import jax.numpy as jnp
from jax import lax
from jax.experimental import pallas as pl
from jax.experimental.pallas import tpu as pltpu
from jax.experimental.pallas import tpu_sc as plsc

_NC = 2
_NS = 16
_NW = _NC * _NS
_LANES = 16
_CHUNK = 128
_ZROWS = 64

_MESH = plsc.VectorSubcoreMesh(core_axis_name="c", subcore_axis_name="s")
_CP_NO_LAYOUT = dataclasses.replace(
    pltpu.CompilerParams(), needs_layout_passes=False)


def _sc_degree(col3, n_pad, cpt):
    @functools.partial(
        pl.kernel,
        out_type=jax.ShapeDtypeStruct((_NW, n_pad), jnp.float32),
        mesh=_MESH,
        compiler_params=_CP_NO_LAYOUT,
        scratch_types=[
            pltpu.VMEM((cpt, _CHUNK), jnp.int32),
            pltpu.VMEM((n_pad,), jnp.float32),
        ],
    )
    def deg_kernel(col_hbm, out_hbm, idx_v, hist_v):
        c = lax.axis_index("c")
        s = lax.axis_index("s")
        wid = c * _NS + s
        pltpu.sync_copy(col_hbm.at[wid], idx_v)

        @pl.loop(0, n_pad, step=_LANES)
        def _(i):
            hist_v[pl.ds(i, _LANES)] = jnp.zeros((_LANES,), jnp.float32)

        ones = jnp.ones((_LANES,), jnp.float32)

        @pl.loop(0, cpt)
        def _(j):
            @pl.loop(0, _CHUNK, step=_LANES)
            def _(q):
                iv = idx_v[j, pl.ds(q, _LANES)]
                plsc.addupdate_scatter(hist_v, [iv], ones)

        pltpu.sync_copy(hist_v, out_hbm.at[wid])

    return deg_kernel(col3)


_NBUF = 2
_SB = 16


def _sc_scatter(y, row3, col3, n_pad, cpt, d):
    rpt = n_pad // _NS
    assert cpt % _SB == 0 and _SB % _NBUF == 0

    @functools.partial(
        pl.kernel,
        out_type=jax.ShapeDtypeStruct((_NC, n_pad, d), jnp.float32),
        mesh=_MESH,
        scratch_types=[
            pltpu.VMEM((_SB, _CHUNK), jnp.int32),
            pltpu.VMEM((_SB, _CHUNK), jnp.int32),
            [pltpu.VMEM((_CHUNK, d), jnp.float32) for _ in range(_NBUF)],
            [pltpu.SemaphoreType.DMA for _ in range(_NBUF)],
            [pltpu.SemaphoreType.DMA for _ in range(_NBUF)],
            pltpu.VMEM((_LANES, d), jnp.float32),
            pltpu.VMEM_SHARED((n_pad, d), jnp.float32),
        ],
    )
    def scat_kernel(y_hbm, row_hbm, col_hbm, out_hbm,
                    ridx, cidx, msgs, sems, ssems, zbuf, acc_sh):
        c = lax.axis_index("c")
        s = lax.axis_index("s")
        wid = c * _NS + s

        @pl.loop(0, _LANES)
        def _(i):
            @pl.loop(0, d, step=_LANES)
            def _(q):
                zbuf[i, pl.ds(q, _LANES)] = jnp.zeros((_LANES,), jnp.float32)

        @pl.loop(0, rpt, step=_LANES)
        def _(r):
            pltpu.sync_copy(zbuf, acc_sh.at[pl.ds(s * rpt + r, _LANES)])

        plsc.subcore_barrier()

        @pl.loop(0, cpt, step=_SB)
        def _(jb):
            pltpu.sync_copy(row_hbm.at[wid, pl.ds(jb, _SB)], ridx)
            pltpu.sync_copy(col_hbm.at[wid, pl.ds(jb, _SB)], cidx)
            for b in range(_NBUF):
                pltpu.async_copy(y_hbm.at[ridx.at[b]], msgs[b], sems[b])

            @pl.loop(0, _SB, step=_NBUF)
            def _(q):
                for b in range(_NBUF):
                    pltpu.make_async_copy(
                        y_hbm.at[ridx.at[q + b]], msgs[b], sems[b]).wait()
                    pltpu.async_copy(msgs[b], acc_sh.at[cidx.at[q + b]],
                                     ssems[b], add=True)

                    @pl.when(q + b + _NBUF < _SB)
                    def _():
                        pltpu.make_async_copy(
                            msgs[b], acc_sh.at[cidx.at[q + b]],
                            ssems[b]).wait()
                        pltpu.async_copy(
                            y_hbm.at[ridx.at[q + b + _NBUF]], msgs[b], sems[b])

            for b in range(_NBUF):
                pltpu.make_async_copy(
                    msgs[b], acc_sh.at[cidx.at[_SB - _NBUF + b]],
                    ssems[b]).wait()

        plsc.subcore_barrier()
        pltpu.sync_copy(acc_sh.at[pl.ds(s * rpt, rpt)],
                        out_hbm.at[c, pl.ds(s * rpt, rpt)])

    return scat_kernel(y, row3, col3)


_BLK = 1024


def _tc_matmul(x, w):
    n, d = x.shape
    blk = _BLK

    def body(x_ref, w_ref, o_ref):
        o_ref[...] = jnp.dot(x_ref[...], w_ref[...],
                             preferred_element_type=jnp.float32)

    return pl.pallas_call(
        body,
        grid=(n // blk,),
        in_specs=[pl.BlockSpec((blk, d), lambda i: (i, 0)),
                  pl.BlockSpec((d, d), lambda i: (0, 0))],
        out_specs=pl.BlockSpec((blk, d), lambda i: (i, 0)),
        out_shape=jax.ShapeDtypeStruct((n, d), jnp.float32),
    )(x, w)


def _tc_scale(xw, degp):
    n, d = xw.shape
    blk = _BLK

    def body(x_ref, dp_ref, o_ref):
        deg = jnp.sum(dp_ref[...], axis=0) + 1.0
        dis = lax.rsqrt(deg)[:, None]
        o_ref[...] = x_ref[...] * dis

    return pl.pallas_call(
        body,
        grid=(n // blk,),
        in_specs=[pl.BlockSpec((blk, d), lambda i: (i, 0)),
                  pl.BlockSpec((_NW, blk), lambda i: (0, i))],
        out_specs=pl.BlockSpec((blk, d), lambda i: (i, 0)),
        out_shape=jax.ShapeDtypeStruct((n, d), jnp.float32),
    )(xw, degp)


def _tc_finalize(accp, y, degp, b, bn_weight, bn_bias, bn_mean, bn_var):
    n, d = y.shape
    blk = _BLK
    p2 = lambda a: a.reshape(1, d)

    def body(a_ref, y_ref, dp_ref, b_ref, w_ref, bb_ref, m_ref, v_ref, o_ref):
        deg = jnp.sum(dp_ref[...], axis=0) + 1.0
        dis = lax.rsqrt(deg)[:, None]
        t = (a_ref[0] + a_ref[1] + y_ref[...]) * dis + b_ref[...]
        t = jnp.maximum(t, 0.0)
        scale = w_ref[...] * lax.rsqrt(v_ref[...] + 1e-5)
        o_ref[...] = (t - m_ref[...]) * scale + bb_ref[...]

    vspec = pl.BlockSpec((1, d), lambda i: (0, 0))
    return pl.pallas_call(
        body,
        grid=(n // blk,),
        in_specs=[pl.BlockSpec((_NC, blk, d), lambda i: (0, i, 0)),
                  pl.BlockSpec((blk, d), lambda i: (i, 0)),
                  pl.BlockSpec((_NW, blk), lambda i: (0, i)),
                  vspec, vspec, vspec, vspec, vspec],
        out_specs=pl.BlockSpec((blk, d), lambda i: (i, 0)),
        out_shape=jax.ShapeDtypeStruct((n, d), jnp.float32),
    )(accp, y, degp, p2(b), p2(bn_weight), p2(bn_bias), p2(bn_mean),
      p2(bn_var))


def kernel(node_features, edge_indices, W, b, bn_weight, bn_bias, bn_mean,
           bn_var):
    n, d = node_features.shape
    e = edge_indices.shape[1]

    rpt = -(-n // _NS)
    rpt += (-rpt) % _ZROWS
    n_pad = _NS * rpt

    cpt = -(-e // (_NW * _CHUNK))
    cpt += (-cpt) % _SB
    e_pad = _NW * _CHUNK * cpt
    pad = e_pad - e

    row = edge_indices[0].astype(jnp.int32)
    col = edge_indices[1].astype(jnp.int32)
    row_p = jnp.concatenate([row, jnp.zeros((pad,), jnp.int32)])
    pad_cols = n + (jnp.arange(pad, dtype=jnp.int32) % (n_pad - n))
    col_p = jnp.concatenate([col, pad_cols])
    row3 = row_p.reshape(_NW, cpt, _CHUNK)
    col3 = col_p.reshape(_NW, cpt, _CHUNK)

    x_pad = jnp.pad(node_features, ((0, n_pad - n), (0, 0)))

    degp = _sc_degree(col3, n_pad, cpt)
    xw = _tc_matmul(x_pad, W)
    y = _tc_scale(xw, degp)
    accp = _sc_scatter(y, row3, col3, n_pad, cpt, d)
    h = _tc_finalize(accp, y, degp, b, bn_weight, bn_bias, bn_mean, bn_var)
    return h[:n]
